# baseline (speedup 1.0000x reference)
import functools

import jax
import jax.numpy as jnp
from jax import lax
from jax.experimental import pallas as pl
from jax.experimental.pallas import tpu as pltpu
from jax.experimental.pallas import tpu_sc as plsc

N = 10000
NPAD = 10240
E = 160000
EB = 128
NCHUNK = 1280
EPAD = NCHUNK * EB
NC, NS = 2, 16
NW = NC * NS
ROWS_W = NCHUNK // NW
EDGES_W = E // NW
STRIPE = NPAD // NS
BLK = 1000
GRID = N // BLK

_mesh = plsc.VectorSubcoreMesh(core_axis_name="c", subcore_axis_name="s")


@functools.partial(
    pl.kernel,
    mesh=_mesh,
    compiler_params=pltpu.CompilerParams(needs_layout_passes=False),
    out_type=jax.ShapeDtypeStruct((NC, NPAD), jnp.float32),
    scratch_types=[
        pltpu.VMEM((EDGES_W + 16,), jnp.int32),
        pltpu.VMEM((NPAD,), jnp.float32),
        pltpu.VMEM((NS, STRIPE), jnp.float32),
        pltpu.VMEM((STRIPE,), jnp.float32),
        pltpu.VMEM_SHARED((NS, NPAD), jnp.float32),
    ],
)
def _deg(dst_hbm, out_hbm, dstl, hist, buf, acc, shared):
    c = lax.axis_index("c")
    s = lax.axis_index("s")
    wid = c * NS + s
    zeros = jnp.zeros((16,), jnp.float32)
    ones = jnp.ones((16,), jnp.float32)
    dstl[pl.ds((EDGES_W // 16) * 16, 16)] = jnp.zeros((16,), jnp.int32)
    pltpu.sync_copy(dst_hbm.at[pl.ds(wid * EDGES_W, EDGES_W)],
                    dstl.at[pl.ds(0, EDGES_W)])

    @pl.loop(0, NPAD // 16)
    def _(i):
        hist[pl.ds(i * 16, 16)] = zeros

    nfull = EDGES_W // 16

    @pl.loop(0, nfull)
    def _(i):
        idx = dstl[pl.ds(i * 16, 16)]
        plsc.addupdate_scatter(hist, [idx], ones)

    tail = EDGES_W - nfull * 16
    if tail:
        idx = dstl[pl.ds(nfull * 16, 16)]
        mask = lax.iota(jnp.int32, 16) < tail
        plsc.addupdate_scatter(hist, [idx], ones, mask=mask)

    pltpu.sync_copy(hist, shared.at[s])
    plsc.subcore_barrier()
    pltpu.sync_copy(shared.at[:, pl.ds(s * STRIPE, STRIPE)], buf)

    @pl.loop(0, STRIPE // 16)
    def _(j):
        v = buf[0, pl.ds(j * 16, 16)]
        for k in range(1, NS):
            v = v + buf[k, pl.ds(j * 16, 16)]
        acc[pl.ds(j * 16, 16)] = v

    pltpu.sync_copy(acc, out_hbm.at[c, pl.ds(s * STRIPE, STRIPE)])


def _make_agg(D):
    @functools.partial(
        pl.kernel,
        mesh=_mesh,
        out_type=jax.ShapeDtypeStruct((NC, NPAD, D), jnp.float32),
        scratch_types=[
            pltpu.VMEM((ROWS_W, EB), jnp.int32),
            pltpu.VMEM((ROWS_W, EB), jnp.int32),
            pltpu.VMEM((EB, D), jnp.float32),
            pltpu.VMEM((EB, D), jnp.float32),
            pltpu.VMEM_SHARED((NPAD, D), jnp.float32),
            pltpu.SemaphoreType.DMA,
            pltpu.SemaphoreType.DMA,
        ],
    )
    def _agg(hs_hbm, src_hbm, dst_hbm, zeros_hbm, out_hbm,
             srcl, dstl, rows0, rows1, aggsh, sem0, sem1):
        c = lax.axis_index("c")
        s = lax.axis_index("s")
        wid = c * NS + s
        pltpu.sync_copy(src_hbm.at[pl.ds(wid * ROWS_W, ROWS_W)], srcl)
        pltpu.sync_copy(dst_hbm.at[pl.ds(wid * ROWS_W, ROWS_W)], dstl)
        pltpu.sync_copy(zeros_hbm, aggsh.at[pl.ds(s * STRIPE, STRIPE)])
        plsc.subcore_barrier()

        pltpu.async_copy(hs_hbm.at[srcl.at[0]], rows0, sem0)
        pltpu.async_copy(hs_hbm.at[srcl.at[1]], rows1, sem1)

        @pl.loop(0, ROWS_W // 2 - 1)
        def _(i):
            j = i * 2
            pltpu.make_async_copy(hs_hbm.at[srcl.at[j]], rows0, sem0).wait()
            pltpu.sync_copy(rows0, aggsh.at[dstl.at[j]], add=True)
            pltpu.async_copy(hs_hbm.at[srcl.at[j + 2]], rows0, sem0)
            pltpu.make_async_copy(
                hs_hbm.at[srcl.at[j + 1]], rows1, sem1).wait()
            pltpu.sync_copy(rows1, aggsh.at[dstl.at[j + 1]], add=True)
            pltpu.async_copy(hs_hbm.at[srcl.at[j + 3]], rows1, sem1)

        pltpu.make_async_copy(
            hs_hbm.at[srcl.at[ROWS_W - 2]], rows0, sem0).wait()
        pltpu.sync_copy(rows0, aggsh.at[dstl.at[ROWS_W - 2]], add=True)
        pltpu.make_async_copy(
            hs_hbm.at[srcl.at[ROWS_W - 1]], rows1, sem1).wait()
        pltpu.sync_copy(rows1, aggsh.at[dstl.at[ROWS_W - 1]], add=True)
        plsc.subcore_barrier()
        pltpu.sync_copy(aggsh.at[pl.ds(s * STRIPE, STRIPE)],
                        out_hbm.at[c, pl.ds(s * STRIPE, STRIPE)])

    return _agg


_agg128 = _make_agg(128)


def _mm_body(x_ref, w_ref, h_ref):
    h_ref[...] = jnp.dot(x_ref[...], w_ref[...],
                         preferred_element_type=jnp.float32)


def _dinv_body(p_ref, o_ref):
    o_ref[...] = lax.rsqrt(1.0 + p_ref[0] + p_ref[1])[:, None]


def _scale_body(h_ref, d_ref, hs_ref):
    hs_ref[...] = h_ref[...] * d_ref[...]


def _fuse1_body(p_ref, hs_ref, d_ref, b1_ref, o_ref):
    t = (p_ref[0] + p_ref[1] + hs_ref[...]) * d_ref[...] + b1_ref[...]
    o_ref[...] = jnp.maximum(t, 0.0) * d_ref[...]


def _fuse2_body(p_ref, g_ref, d_ref, w2_ref, b2_ref, o_ref):
    v = (p_ref[0] + p_ref[1] + g_ref[...]) * d_ref[...]
    o_ref[...] = (
        jnp.dot(v, w2_ref[...], preferred_element_type=jnp.float32)
        + b2_ref[...]
    )


def kernel(x, edge_index, W1, b1, W2, b2):
    f32 = jnp.float32
    ei = edge_index.astype(jnp.int32)
    src2d = jnp.pad(ei[0], (0, EPAD - E)).reshape(NCHUNK, EB)
    dpad = N + (jnp.arange(EPAD - E, dtype=jnp.int32) % (NPAD - N))
    dst2d = jnp.concatenate([ei[1], dpad]).reshape(NCHUNK, EB)
    b1r = b1.reshape(1, -1)
    b2r = b2.reshape(1, -1)
    z128 = jnp.zeros((STRIPE, 128), f32)

    parts_deg = _deg(ei[1])

    h1 = pl.pallas_call(
        _mm_body,
        grid=(GRID,),
        in_specs=[
            pl.BlockSpec((BLK, 256), lambda i: (i, 0)),
            pl.BlockSpec((256, 128), lambda i: (0, 0)),
        ],
        out_specs=pl.BlockSpec((BLK, 128), lambda i: (i, 0)),
        out_shape=jax.ShapeDtypeStruct((N, 128), f32),
    )(x, W1)

    dinv = pl.pallas_call(
        _dinv_body,
        out_shape=jax.ShapeDtypeStruct((NPAD, 1), f32),
    )(parts_deg)

    hs1 = pl.pallas_call(
        _scale_body,
        grid=(GRID,),
        in_specs=[
            pl.BlockSpec((BLK, 128), lambda i: (i, 0)),
            pl.BlockSpec((BLK, 1), lambda i: (i, 0)),
        ],
        out_specs=pl.BlockSpec((BLK, 128), lambda i: (i, 0)),
        out_shape=jax.ShapeDtypeStruct((N, 128), f32),
    )(h1, dinv)

    parts1 = _agg128(hs1, src2d, dst2d, z128)

    g = pl.pallas_call(
        _fuse1_body,
        grid=(GRID,),
        in_specs=[
            pl.BlockSpec((NC, BLK, 128), lambda i: (0, i, 0)),
            pl.BlockSpec((BLK, 128), lambda i: (i, 0)),
            pl.BlockSpec((BLK, 1), lambda i: (i, 0)),
            pl.BlockSpec((1, 128), lambda i: (0, 0)),
        ],
        out_specs=pl.BlockSpec((BLK, 128), lambda i: (i, 0)),
        out_shape=jax.ShapeDtypeStruct((N, 128), f32),
    )(parts1, hs1, dinv, b1r)

    parts2 = _agg128(g, src2d, dst2d, z128)

    out = pl.pallas_call(
        _fuse2_body,
        grid=(GRID,),
        in_specs=[
            pl.BlockSpec((NC, BLK, 128), lambda i: (0, i, 0)),
            pl.BlockSpec((BLK, 128), lambda i: (i, 0)),
            pl.BlockSpec((BLK, 1), lambda i: (i, 0)),
            pl.BlockSpec((128, 10), lambda i: (0, 0)),
            pl.BlockSpec((1, 10), lambda i: (0, 0)),
        ],
        out_specs=pl.BlockSpec((BLK, 10), lambda i: (i, 0)),
        out_shape=jax.ShapeDtypeStruct((N, 10), f32),
    )(parts2, g, dinv, W2, b2r)

    return out


# speedup vs baseline: 2.3037x; 2.3037x over previous
import functools

import jax
import jax.numpy as jnp
from jax import lax
from jax.experimental import pallas as pl
from jax.experimental.pallas import tpu as pltpu
from jax.experimental.pallas import tpu_sc as plsc

N = 10000
NPAD = 10240
E = 160000
EB = 125
NCHUNK = E // EB
NC, NS = 2, 16
NW = NC * NS
ROWS_W = NCHUNK // NW
EDGES_W = E // NW
STRIPE = NPAD // NS
BLK = 1000
GRID = N // BLK

_mesh = plsc.VectorSubcoreMesh(core_axis_name="c", subcore_axis_name="s")


@functools.partial(
    pl.kernel,
    mesh=_mesh,
    compiler_params=pltpu.CompilerParams(needs_layout_passes=False),
    out_type=jax.ShapeDtypeStruct((NC, NPAD), jnp.float32),
    scratch_types=[
        pltpu.VMEM((EDGES_W + 16,), jnp.int32),
        pltpu.VMEM((NPAD,), jnp.float32),
        pltpu.VMEM((NS, STRIPE), jnp.float32),
        pltpu.VMEM((STRIPE,), jnp.float32),
        pltpu.VMEM_SHARED((NS, NPAD), jnp.float32),
    ],
)
def _deg(dst_hbm, out_hbm, dstl, hist, buf, acc, shared):
    c = lax.axis_index("c")
    s = lax.axis_index("s")
    wid = c * NS + s
    zeros = jnp.zeros((16,), jnp.float32)
    ones = jnp.ones((16,), jnp.float32)
    dstl[pl.ds((EDGES_W // 16) * 16, 16)] = jnp.zeros((16,), jnp.int32)
    pltpu.sync_copy(dst_hbm.at[pl.ds(wid * EDGES_W, EDGES_W)],
                    dstl.at[pl.ds(0, EDGES_W)])

    @pl.loop(0, NPAD // 16)
    def _(i):
        hist[pl.ds(i * 16, 16)] = zeros

    nfull = EDGES_W // 16

    @pl.loop(0, nfull)
    def _(i):
        idx = dstl[pl.ds(i * 16, 16)]
        plsc.addupdate_scatter(hist, [idx], ones)

    tail = EDGES_W - nfull * 16
    if tail:
        idx = dstl[pl.ds(nfull * 16, 16)]
        mask = lax.iota(jnp.int32, 16) < tail
        plsc.addupdate_scatter(hist, [idx], ones, mask=mask)

    pltpu.sync_copy(hist, shared.at[s])
    plsc.subcore_barrier()
    pltpu.sync_copy(shared.at[:, pl.ds(s * STRIPE, STRIPE)], buf)

    @pl.loop(0, STRIPE // 16)
    def _(j):
        v = buf[0, pl.ds(j * 16, 16)]
        for k in range(1, NS):
            v = v + buf[k, pl.ds(j * 16, 16)]
        acc[pl.ds(j * 16, 16)] = v

    pltpu.sync_copy(acc, out_hbm.at[c, pl.ds(s * STRIPE, STRIPE)])


def _make_agg(D):
    @functools.partial(
        pl.kernel,
        mesh=_mesh,
        out_type=jax.ShapeDtypeStruct((NC, NPAD, D), jnp.float32),
        scratch_types=[
            pltpu.VMEM((ROWS_W, EB), jnp.int32),
            pltpu.VMEM((ROWS_W, EB), jnp.int32),
            pltpu.VMEM((EB, D), jnp.float32),
            pltpu.VMEM((EB, D), jnp.float32),
            pltpu.VMEM_SHARED((NPAD, D), jnp.float32),
            pltpu.SemaphoreType.DMA,
            pltpu.SemaphoreType.DMA,
        ],
    )
    def _agg(hs_hbm, src_hbm, dst_hbm, zeros_hbm, out_hbm,
             srcl, dstl, rows0, rows1, aggsh, sem0, sem1):
        c = lax.axis_index("c")
        s = lax.axis_index("s")
        wid = c * NS + s
        pltpu.sync_copy(src_hbm.at[pl.ds(wid * ROWS_W, ROWS_W)], srcl)
        pltpu.sync_copy(dst_hbm.at[pl.ds(wid * ROWS_W, ROWS_W)], dstl)
        pltpu.sync_copy(zeros_hbm, aggsh.at[pl.ds(s * STRIPE, STRIPE)])
        plsc.subcore_barrier()

        pltpu.async_copy(hs_hbm.at[srcl.at[0]], rows0, sem0)
        pltpu.async_copy(hs_hbm.at[srcl.at[1]], rows1, sem1)

        @pl.loop(0, ROWS_W // 2 - 1)
        def _(i):
            j = i * 2
            pltpu.make_async_copy(hs_hbm.at[srcl.at[j]], rows0, sem0).wait()
            pltpu.sync_copy(rows0, aggsh.at[dstl.at[j]], add=True)
            pltpu.async_copy(hs_hbm.at[srcl.at[j + 2]], rows0, sem0)
            pltpu.make_async_copy(
                hs_hbm.at[srcl.at[j + 1]], rows1, sem1).wait()
            pltpu.sync_copy(rows1, aggsh.at[dstl.at[j + 1]], add=True)
            pltpu.async_copy(hs_hbm.at[srcl.at[j + 3]], rows1, sem1)

        pltpu.make_async_copy(
            hs_hbm.at[srcl.at[ROWS_W - 2]], rows0, sem0).wait()
        pltpu.sync_copy(rows0, aggsh.at[dstl.at[ROWS_W - 2]], add=True)
        pltpu.make_async_copy(
            hs_hbm.at[srcl.at[ROWS_W - 1]], rows1, sem1).wait()
        pltpu.sync_copy(rows1, aggsh.at[dstl.at[ROWS_W - 1]], add=True)
        plsc.subcore_barrier()
        pltpu.sync_copy(aggsh.at[pl.ds(s * STRIPE, STRIPE)],
                        out_hbm.at[c, pl.ds(s * STRIPE, STRIPE)])

    return _agg


_agg128 = _make_agg(128)


def _mm_body(x_ref, w_ref, h_ref):
    h_ref[...] = jnp.dot(x_ref[...], w_ref[...],
                         preferred_element_type=jnp.float32)


def _dinv_body(p_ref, o_ref):
    o_ref[...] = lax.rsqrt(1.0 + p_ref[0] + p_ref[1])[:, None]


def _scale_body(h_ref, d_ref, hs_ref):
    hs_ref[...] = h_ref[...] * d_ref[...]


def _fuse1_body(p_ref, hs_ref, d_ref, b1_ref, o_ref):
    t = (p_ref[0] + p_ref[1] + hs_ref[...]) * d_ref[...] + b1_ref[...]
    o_ref[...] = jnp.maximum(t, 0.0) * d_ref[...]


def _fuse2_body(p_ref, g_ref, d_ref, w2_ref, b2_ref, o_ref):
    v = (p_ref[0] + p_ref[1] + g_ref[...]) * d_ref[...]
    o_ref[...] = (
        jnp.dot(v, w2_ref[...], preferred_element_type=jnp.float32)
        + b2_ref[...]
    )


def kernel(x, edge_index, W1, b1, W2, b2):
    f32 = jnp.float32
    ei = edge_index.astype(jnp.int32)
    src2d = ei[0].reshape(NCHUNK, EB)
    dst2d = ei[1].reshape(NCHUNK, EB)
    b1r = b1.reshape(1, -1)
    b2r = b2.reshape(1, -1)
    z128 = jnp.zeros((STRIPE, 128), f32)

    parts_deg = _deg(ei[1])

    h1 = pl.pallas_call(
        _mm_body,
        grid=(GRID,),
        in_specs=[
            pl.BlockSpec((BLK, 256), lambda i: (i, 0)),
            pl.BlockSpec((256, 128), lambda i: (0, 0)),
        ],
        out_specs=pl.BlockSpec((BLK, 128), lambda i: (i, 0)),
        out_shape=jax.ShapeDtypeStruct((N, 128), f32),
    )(x, W1)

    dinv = pl.pallas_call(
        _dinv_body,
        out_shape=jax.ShapeDtypeStruct((NPAD, 1), f32),
    )(parts_deg)

    hs1 = pl.pallas_call(
        _scale_body,
        grid=(GRID,),
        in_specs=[
            pl.BlockSpec((BLK, 128), lambda i: (i, 0)),
            pl.BlockSpec((BLK, 1), lambda i: (i, 0)),
        ],
        out_specs=pl.BlockSpec((BLK, 128), lambda i: (i, 0)),
        out_shape=jax.ShapeDtypeStruct((N, 128), f32),
    )(h1, dinv)

    parts1 = _agg128(hs1, src2d, dst2d, z128)

    g = pl.pallas_call(
        _fuse1_body,
        grid=(GRID,),
        in_specs=[
            pl.BlockSpec((NC, BLK, 128), lambda i: (0, i, 0)),
            pl.BlockSpec((BLK, 128), lambda i: (i, 0)),
            pl.BlockSpec((BLK, 1), lambda i: (i, 0)),
            pl.BlockSpec((1, 128), lambda i: (0, 0)),
        ],
        out_specs=pl.BlockSpec((BLK, 128), lambda i: (i, 0)),
        out_shape=jax.ShapeDtypeStruct((N, 128), f32),
    )(parts1, hs1, dinv, b1r)

    parts2 = _agg128(g, src2d, dst2d, z128)

    out = pl.pallas_call(
        _fuse2_body,
        grid=(GRID,),
        in_specs=[
            pl.BlockSpec((NC, BLK, 128), lambda i: (0, i, 0)),
            pl.BlockSpec((BLK, 128), lambda i: (i, 0)),
            pl.BlockSpec((BLK, 1), lambda i: (i, 0)),
            pl.BlockSpec((128, 10), lambda i: (0, 0)),
            pl.BlockSpec((1, 10), lambda i: (0, 0)),
        ],
        out_specs=pl.BlockSpec((BLK, 10), lambda i: (i, 0)),
        out_shape=jax.ShapeDtypeStruct((N, 10), f32),
    )(parts2, g, dinv, W2, b2r)

    return out


# speedup vs baseline: 2.4168x; 1.0491x over previous
import functools

import jax
import jax.numpy as jnp
from jax import lax
from jax.experimental import pallas as pl
from jax.experimental.pallas import tpu as pltpu
from jax.experimental.pallas import tpu_sc as plsc

N = 10000
NPAD = 10240
E = 160000
EB = 125
NCHUNK = E // EB
NC, NS = 2, 16
NW = NC * NS
ROWS_W = NCHUNK // NW
EDGES_W = E // NW
STRIPE = NPAD // NS
BLK = 1000
GRID = N // BLK
MBLK = 1024
MMGRID = NPAD // MBLK

_mesh = plsc.VectorSubcoreMesh(core_axis_name="c", subcore_axis_name="s")


@functools.partial(
    pl.kernel,
    mesh=_mesh,
    compiler_params=pltpu.CompilerParams(needs_layout_passes=False),
    out_type=jax.ShapeDtypeStruct((NC, NPAD), jnp.float32),
    scratch_types=[
        pltpu.VMEM((EDGES_W + 16,), jnp.int32),
        pltpu.VMEM((NPAD,), jnp.float32),
        pltpu.VMEM((NS, STRIPE), jnp.float32),
        pltpu.VMEM((STRIPE,), jnp.float32),
        pltpu.VMEM_SHARED((NS, NPAD), jnp.float32),
    ],
)
def _deg(dst_hbm, out_hbm, dstl, hist, buf, acc, shared):
    c = lax.axis_index("c")
    s = lax.axis_index("s")
    wid = c * NS + s
    zeros = jnp.zeros((16,), jnp.float32)
    ones = jnp.ones((16,), jnp.float32)
    dstl[pl.ds((EDGES_W // 16) * 16, 16)] = jnp.zeros((16,), jnp.int32)
    pltpu.sync_copy(dst_hbm.at[pl.ds(wid * EDGES_W, EDGES_W)],
                    dstl.at[pl.ds(0, EDGES_W)])

    @pl.loop(0, NPAD // 16)
    def _(i):
        hist[pl.ds(i * 16, 16)] = zeros

    nfull = EDGES_W // 16

    @pl.loop(0, nfull)
    def _(i):
        idx = dstl[pl.ds(i * 16, 16)]
        plsc.addupdate_scatter(hist, [idx], ones)

    tail = EDGES_W - nfull * 16
    if tail:
        idx = dstl[pl.ds(nfull * 16, 16)]
        mask = lax.iota(jnp.int32, 16) < tail
        plsc.addupdate_scatter(hist, [idx], ones, mask=mask)

    pltpu.sync_copy(hist, shared.at[s])
    plsc.subcore_barrier()
    pltpu.sync_copy(shared.at[:, pl.ds(s * STRIPE, STRIPE)], buf)

    @pl.loop(0, STRIPE // 16)
    def _(j):
        v = buf[0, pl.ds(j * 16, 16)]
        for k in range(1, NS):
            v = v + buf[k, pl.ds(j * 16, 16)]
        acc[pl.ds(j * 16, 16)] = v

    pltpu.sync_copy(acc, out_hbm.at[c, pl.ds(s * STRIPE, STRIPE)])


def _make_agg(D):
    @functools.partial(
        pl.kernel,
        mesh=_mesh,
        out_type=jax.ShapeDtypeStruct((NC, NPAD, D), jnp.float32),
        scratch_types=[
            pltpu.VMEM((ROWS_W, EB), jnp.int32),
            pltpu.VMEM((ROWS_W, EB), jnp.int32),
            pltpu.VMEM((EB, D), jnp.float32),
            pltpu.VMEM((EB, D), jnp.float32),
            pltpu.VMEM_SHARED((NPAD, D), jnp.float32),
            pltpu.SemaphoreType.DMA,
            pltpu.SemaphoreType.DMA,
        ],
    )
    def _agg(hs_hbm, src_hbm, dst_hbm, zeros_hbm, out_hbm,
             srcl, dstl, rows0, rows1, aggsh, sem0, sem1):
        c = lax.axis_index("c")
        s = lax.axis_index("s")
        wid = c * NS + s
        pltpu.sync_copy(src_hbm.at[pl.ds(wid * ROWS_W, ROWS_W)], srcl)
        pltpu.sync_copy(dst_hbm.at[pl.ds(wid * ROWS_W, ROWS_W)], dstl)
        pltpu.sync_copy(zeros_hbm, aggsh.at[pl.ds(s * STRIPE, STRIPE)])
        plsc.subcore_barrier()

        pltpu.async_copy(hs_hbm.at[srcl.at[0]], rows0, sem0)
        pltpu.async_copy(hs_hbm.at[srcl.at[1]], rows1, sem1)

        @pl.loop(0, ROWS_W // 2 - 1)
        def _(i):
            j = i * 2
            pltpu.make_async_copy(hs_hbm.at[srcl.at[j]], rows0, sem0).wait()
            pltpu.sync_copy(rows0, aggsh.at[dstl.at[j]], add=True)
            pltpu.async_copy(hs_hbm.at[srcl.at[j + 2]], rows0, sem0)
            pltpu.make_async_copy(
                hs_hbm.at[srcl.at[j + 1]], rows1, sem1).wait()
            pltpu.sync_copy(rows1, aggsh.at[dstl.at[j + 1]], add=True)
            pltpu.async_copy(hs_hbm.at[srcl.at[j + 3]], rows1, sem1)

        pltpu.make_async_copy(
            hs_hbm.at[srcl.at[ROWS_W - 2]], rows0, sem0).wait()
        pltpu.sync_copy(rows0, aggsh.at[dstl.at[ROWS_W - 2]], add=True)
        pltpu.make_async_copy(
            hs_hbm.at[srcl.at[ROWS_W - 1]], rows1, sem1).wait()
        pltpu.sync_copy(rows1, aggsh.at[dstl.at[ROWS_W - 1]], add=True)
        plsc.subcore_barrier()
        pltpu.sync_copy(aggsh.at[pl.ds(s * STRIPE, STRIPE)],
                        out_hbm.at[c, pl.ds(s * STRIPE, STRIPE)])

    return _agg


_agg128 = _make_agg(128)


def _mm1_body(p_ref, x_ref, w_ref, hs_ref, d_ref):
    dinv = lax.rsqrt(1.0 + p_ref[0] + p_ref[1])[:, None]
    d_ref[...] = dinv
    h = jnp.dot(x_ref[...], w_ref[...], preferred_element_type=jnp.float32)
    hs_ref[...] = h * dinv


def _fuse1_body(p_ref, hs_ref, d_ref, b1_ref, o_ref):
    t = (p_ref[0] + p_ref[1] + hs_ref[...]) * d_ref[...] + b1_ref[...]
    o_ref[...] = jnp.maximum(t, 0.0) * d_ref[...]


def _fuse2_body(p_ref, g_ref, d_ref, w2_ref, b2_ref, o_ref):
    v = (p_ref[0] + p_ref[1] + g_ref[...]) * d_ref[...]
    o_ref[...] = (
        jnp.dot(v, w2_ref[...], preferred_element_type=jnp.float32)
        + b2_ref[...]
    )


def kernel(x, edge_index, W1, b1, W2, b2):
    f32 = jnp.float32
    ei = edge_index.astype(jnp.int32)
    src2d = ei[0].reshape(NCHUNK, EB)
    dst2d = ei[1].reshape(NCHUNK, EB)
    b1r = b1.reshape(1, -1)
    b2r = b2.reshape(1, -1)
    z128 = jnp.zeros((STRIPE, 128), f32)

    parts_deg = _deg(ei[1])

    hs1, dinv = pl.pallas_call(
        _mm1_body,
        grid=(MMGRID,),
        in_specs=[
            pl.BlockSpec((NC, MBLK), lambda i: (0, i)),
            pl.BlockSpec((MBLK, 256), lambda i: (i, 0)),
            pl.BlockSpec((256, 128), lambda i: (0, 0)),
        ],
        out_specs=(
            pl.BlockSpec((MBLK, 128), lambda i: (i, 0)),
            pl.BlockSpec((MBLK, 1), lambda i: (i, 0)),
        ),
        out_shape=(
            jax.ShapeDtypeStruct((N, 128), f32),
            jax.ShapeDtypeStruct((N, 1), f32),
        ),
    )(parts_deg, x, W1)

    parts1 = _agg128(hs1, src2d, dst2d, z128)

    g = pl.pallas_call(
        _fuse1_body,
        grid=(GRID,),
        in_specs=[
            pl.BlockSpec((NC, BLK, 128), lambda i: (0, i, 0)),
            pl.BlockSpec((BLK, 128), lambda i: (i, 0)),
            pl.BlockSpec((BLK, 1), lambda i: (i, 0)),
            pl.BlockSpec((1, 128), lambda i: (0, 0)),
        ],
        out_specs=pl.BlockSpec((BLK, 128), lambda i: (i, 0)),
        out_shape=jax.ShapeDtypeStruct((N, 128), f32),
    )(parts1, hs1, dinv, b1r)

    parts2 = _agg128(g, src2d, dst2d, z128)

    out = pl.pallas_call(
        _fuse2_body,
        grid=(GRID,),
        in_specs=[
            pl.BlockSpec((NC, BLK, 128), lambda i: (0, i, 0)),
            pl.BlockSpec((BLK, 128), lambda i: (i, 0)),
            pl.BlockSpec((BLK, 1), lambda i: (i, 0)),
            pl.BlockSpec((128, 10), lambda i: (0, 0)),
            pl.BlockSpec((1, 10), lambda i: (0, 0)),
        ],
        out_specs=pl.BlockSpec((BLK, 10), lambda i: (i, 0)),
        out_shape=jax.ShapeDtypeStruct((N, 10), f32),
    )(parts2, g, dinv, W2, b2r)

    return out
